# Initial kernel scaffold; baseline (speedup 1.0000x reference)
import jax
import jax.numpy as jnp
from jax.experimental import pallas as pl


def kernel(x, edge_index, W1, b1, Wmu, bmu, Wls, bls):
    raise NotImplementedError("write your pallas kernel here")


# speedup vs baseline: 14.2664x; 14.2664x over previous
import functools

import jax
import jax.numpy as jnp
from jax import lax
from jax.experimental import pallas as pl
from jax.experimental.pallas import tpu as pltpu
from jax.experimental.pallas import tpu_sc as plsc

NC = 2
NS = 16
LANES = 16

CHUNK = 80
IROWS = 25


def _deg_kernel(n_pad, n_edges):
  ept = n_edges // (NC * NS)
  n_outer = ept // (CHUNK * IROWS)
  rpt = n_pad // NS
  mesh = plsc.VectorSubcoreMesh(core_axis_name="c", subcore_axis_name="s")

  def body(dst_hbm, zero_hbm, out_hbm, dst_v, ones_v, acc_sh):
    cid = lax.axis_index("c")
    sid = lax.axis_index("s")
    r0 = sid * rpt
    pltpu.sync_copy(zero_hbm.at[pl.ds(r0, rpt)], acc_sh.at[pl.ds(r0, rpt)])
    for k in range(CHUNK // LANES):
      ones_v[pl.ds(k * LANES, LANES)] = jnp.full((LANES,), 1.0, jnp.float32)
    plsc.subcore_barrier()

    row0 = (cid * NS + sid) * (ept // CHUNK)

    def outer(i, carry):
      pltpu.sync_copy(dst_hbm.at[pl.ds(row0 + i * IROWS, IROWS)], dst_v)

      def inner(j, c2):
        pltpu.sync_copy(ones_v, acc_sh.at[dst_v.at[j]], add=True)
        return c2

      return lax.fori_loop(0, IROWS, inner, carry)

    lax.fori_loop(0, n_outer, outer, 0)
    plsc.subcore_barrier()
    pltpu.sync_copy(acc_sh.at[pl.ds(r0, rpt)],
                    out_hbm.at[cid].at[pl.ds(r0, rpt)])

  return pl.kernel(
      body,
      out_type=jax.ShapeDtypeStruct((NC, n_pad), jnp.float32),
      mesh=mesh,
      compiler_params=pltpu.CompilerParams(use_tc_tiling_on_sc=False),
      scratch_types=[
          pltpu.VMEM((IROWS, CHUNK), jnp.int32),
          pltpu.VMEM((CHUNK,), jnp.float32),
          pltpu.VMEM_SHARED((n_pad,), jnp.float32),
      ],
  )


def _agg_kernel(n_nodes, n_pad, n_edges, d_half):
  ept = n_edges // NS
  n_outer = ept // (CHUNK * IROWS)
  rpt = n_pad // NS
  mesh = plsc.VectorSubcoreMesh(core_axis_name="c", subcore_axis_name="s")

  def body(t_hbm, src_hbm, dst_hbm, zero_hbm, out_hbm,
           src_v, dst_v, rows_v, acc_sh):
    cid = lax.axis_index("c")
    sid = lax.axis_index("s")
    r0 = sid * rpt
    pltpu.sync_copy(zero_hbm.at[pl.ds(r0, rpt)], acc_sh.at[pl.ds(r0, rpt)])
    plsc.subcore_barrier()

    off = cid * n_nodes
    row0 = sid * (ept // CHUNK)

    def outer(i, carry):
      pltpu.sync_copy(src_hbm.at[pl.ds(row0 + i * IROWS, IROWS)], src_v)
      pltpu.sync_copy(dst_hbm.at[pl.ds(row0 + i * IROWS, IROWS)], dst_v)

      def shift(j, c2):
        def shift16(k, c3):
          sl = (j, pl.ds(k * LANES, LANES))
          src_v[sl] = src_v[sl] + off
          return c3
        return lax.fori_loop(0, CHUNK // LANES, shift16, c2)

      lax.fori_loop(0, IROWS, shift, 0)

      def inner(j, c2):
        pltpu.sync_copy(t_hbm.at[src_v.at[j]], rows_v)
        pltpu.sync_copy(rows_v, acc_sh.at[dst_v.at[j]], add=True)
        return c2

      return lax.fori_loop(0, IROWS, inner, carry)

    lax.fori_loop(0, n_outer, outer, 0)
    plsc.subcore_barrier()
    pltpu.sync_copy(acc_sh.at[pl.ds(r0, rpt)],
                    out_hbm.at[cid].at[pl.ds(r0, rpt)])

  return pl.kernel(
      body,
      out_type=jax.ShapeDtypeStruct((NC, n_pad, d_half), jnp.float32),
      mesh=mesh,
      compiler_params=pltpu.CompilerParams(use_tc_tiling_on_sc=False),
      scratch_types=[
          pltpu.VMEM((IROWS, CHUNK), jnp.int32),
          pltpu.VMEM((IROWS, CHUNK), jnp.int32),
          pltpu.VMEM((CHUNK, d_half), jnp.float32),
          pltpu.VMEM_SHARED((n_pad, d_half), jnp.float32),
      ],
  )


_RB = 1000


def _tc_first(n, d_in, d_hid):
  d_half = d_hid // 2

  def body(x_ref, w_ref, dp_ref, out_ref):
    dinv = lax.rsqrt(dp_ref[0] + dp_ref[1] + 1.0)
    t = jnp.dot(x_ref[...], w_ref[...],
                preferred_element_type=jnp.float32) * dinv
    out_ref[0] = t[:, :d_half]
    out_ref[1] = t[:, d_half:]

  return pl.pallas_call(
      body,
      grid=(n // _RB,),
      in_specs=[
          pl.BlockSpec((_RB, d_in), lambda i: (i, 0)),
          pl.BlockSpec((d_in, d_hid), lambda i: (0, 0)),
          pl.BlockSpec((2, _RB, 1), lambda i: (0, i, 0)),
      ],
      out_specs=pl.BlockSpec((2, _RB, d_half), lambda i: (0, i, 0)),
      out_shape=jax.ShapeDtypeStruct((2, n, d_half), jnp.float32),
  )


def _tc_mid(n, d_hid):
  d_half = d_hid // 2

  def body(s_ref, t_ref, dp_ref, b_ref, w_ref, out_ref):
    dinv = lax.rsqrt(dp_ref[0] + dp_ref[1] + 1.0)
    h0 = jnp.maximum(dinv * (s_ref[0] + t_ref[0]) + b_ref[0, :d_half], 0.0)
    h1 = jnp.maximum(dinv * (s_ref[1] + t_ref[1]) + b_ref[0, d_half:], 0.0)
    u = (jnp.dot(h0, w_ref[:d_half, :], preferred_element_type=jnp.float32)
         + jnp.dot(h1, w_ref[d_half:, :], preferred_element_type=jnp.float32))
    u = u * dinv
    out_ref[0] = u[:, :d_half]
    out_ref[1] = u[:, d_half:]

  return pl.pallas_call(
      body,
      grid=(n // _RB,),
      in_specs=[
          pl.BlockSpec((2, _RB, d_half), lambda i: (0, i, 0)),
          pl.BlockSpec((2, _RB, d_half), lambda i: (0, i, 0)),
          pl.BlockSpec((2, _RB, 1), lambda i: (0, i, 0)),
          pl.BlockSpec((1, d_hid), lambda i: (0, 0)),
          pl.BlockSpec((d_hid, d_hid), lambda i: (0, 0)),
      ],
      out_specs=pl.BlockSpec((2, _RB, d_half), lambda i: (0, i, 0)),
      out_shape=jax.ShapeDtypeStruct((2, n, d_half), jnp.float32),
  )


def _tc_final(n, d_out):
  def body(s_ref, u_ref, dp_ref, bmu_ref, bls_ref, mu_ref, ls_ref):
    dinv = lax.rsqrt(dp_ref[0] + dp_ref[1] + 1.0)
    mu_ref[...] = dinv * (s_ref[0] + u_ref[0]) + bmu_ref[0]
    ls_ref[...] = dinv * (s_ref[1] + u_ref[1]) + bls_ref[0]

  return pl.pallas_call(
      body,
      grid=(n // _RB,),
      in_specs=[
          pl.BlockSpec((2, _RB, d_out), lambda i: (0, i, 0)),
          pl.BlockSpec((2, _RB, d_out), lambda i: (0, i, 0)),
          pl.BlockSpec((2, _RB, 1), lambda i: (0, i, 0)),
          pl.BlockSpec((1, d_out), lambda i: (0, 0)),
          pl.BlockSpec((1, d_out), lambda i: (0, 0)),
      ],
      out_specs=[
          pl.BlockSpec((_RB, d_out), lambda i: (i, 0)),
          pl.BlockSpec((_RB, d_out), lambda i: (i, 0)),
      ],
      out_shape=[
          jax.ShapeDtypeStruct((n, d_out), jnp.float32),
          jax.ShapeDtypeStruct((n, d_out), jnp.float32),
      ],
  )


@jax.jit
def kernel(x, edge_index, W1, b1, Wmu, bmu, Wls, bls):
  n, d_in = x.shape
  e = edge_index.shape[1]
  d_hid = W1.shape[1]
  d_half = d_hid // 2
  d_out = Wmu.shape[1]
  assert n % _RB == 0 and e % (NC * NS * CHUNK * IROWS) == 0
  n_pad = -(-n // (NS * 8)) * (NS * 8)

  src2d = edge_index[0].reshape(e // CHUNK, CHUNK)
  dst2d = edge_index[1].reshape(e // CHUNK, CHUNK)
  zeros2d = jnp.zeros((n_pad, d_half), jnp.float32)
  zeros1d = jnp.zeros((n_pad,), jnp.float32)

  degp = _deg_kernel(n_pad, e)(dst2d, zeros1d)
  degp3 = degp[:, :n, None]

  tp = _tc_first(n, d_in, d_hid)(x, W1, degp3)
  s1 = _agg_kernel(n, n_pad, e, d_half)(
      tp.reshape(2 * n, d_half), src2d, dst2d, zeros2d)[:, :n, :]

  wcat = jnp.concatenate([Wmu, Wls], axis=1)
  up = _tc_mid(n, d_hid)(s1, tp, degp3, b1.reshape(1, d_hid), wcat)
  s2 = _agg_kernel(n, n_pad, e, d_half)(
      up.reshape(2 * n, d_half), src2d, dst2d, zeros2d)[:, :n, :]

  mu, ls = _tc_final(n, d_out)(s2, up, degp3,
                               bmu.reshape(1, d_out), bls.reshape(1, d_out))
  return (mu, ls)
